# baseline (speedup 1.0000x reference)
import functools

import jax
import jax.numpy as jnp
from jax import lax
from jax.experimental import pallas as pl
from jax.experimental.pallas import tpu as pltpu
from jax.experimental.pallas import tpu_sc as plsc

_B, _L, _EMB = 4096, 100, 768
_PROMPT = 100
_VPAD = 128
_NC, _NS = 2, 16
_NW = _NC * _NS
_N = _B * _L
_PER_W = _N // _NW
_C = 80
_NCH = _PER_W // _C


def _y_body(t_ref, w_ref, b_ref, y_ref):
    h = lax.dot_general(t_ref[...], w_ref[...],
                        (((1,), (1,)), ((), ())),
                        preferred_element_type=jnp.float32)
    h = h + b_ref[...]
    y_ref[...] = 0.5 * h * (1.0 + lax.erf(h * 0.7071067811865476))


def _fold_table(table, W, b):
    tp = jnp.zeros((_VPAD, _EMB), jnp.float32).at[:_PROMPT, :].set(table)
    return pl.pallas_call(
        _y_body,
        out_shape=jax.ShapeDtypeStruct((_VPAD, _EMB), jnp.float32),
    )(tp, W, b.reshape(1, _EMB))


def _gather_body(y_hbm, x_hbm, out_hbm,
                 idx0, idx1, rows0, rows1, gsem0, gsem1, wsem0, wsem1):
    wid = lax.axis_index("s") * _NC + lax.axis_index("c")
    base = wid * _PER_W

    def _load_idx(c, idx_v):
        pltpu.sync_copy(x_hbm.at[pl.ds(base + c * _C, _C)], idx_v)

    def _gather(idx_v, rows, sem):
        pltpu.async_copy(y_hbm.at[idx_v], rows, sem)

    def _wait_gather(idx_v, rows, sem):
        pltpu.make_async_copy(y_hbm.at[idx_v], rows, sem).wait()

    def _dst(c):
        return out_hbm.at[pl.ds(base + c * _C, _C)]

    _load_idx(0, idx0)
    _gather(idx0, rows0, gsem0)

    @pl.loop(0, _NCH, step=2)
    def _pair(c):
        @pl.when(c > 0)
        def _():
            pltpu.make_async_copy(rows1, _dst(c - 1), wsem1).wait()

        _load_idx(c + 1, idx1)
        _gather(idx1, rows1, gsem1)

        _wait_gather(idx0, rows0, gsem0)
        pltpu.async_copy(rows0, _dst(c), wsem0)

        _wait_gather(idx1, rows1, gsem1)
        pltpu.async_copy(rows1, _dst(c + 1), wsem1)

        pltpu.make_async_copy(rows0, _dst(c), wsem0).wait()

        @pl.when(c + 2 < _NCH)
        def _():
            _load_idx(c + 2, idx0)
            _gather(idx0, rows0, gsem0)

    pltpu.make_async_copy(rows1, _dst(_NCH - 1), wsem1).wait()


_gather_call = functools.partial(
    pl.kernel,
    out_type=jax.ShapeDtypeStruct((_N, _EMB), jnp.float32),
    mesh=plsc.VectorSubcoreMesh(core_axis_name="c", subcore_axis_name="s"),
    scratch_types=[
        pltpu.VMEM((_C,), jnp.int32),
        pltpu.VMEM((_C,), jnp.int32),
        pltpu.VMEM((_C, _EMB), jnp.float32),
        pltpu.VMEM((_C, _EMB), jnp.float32),
        pltpu.SemaphoreType.DMA,
        pltpu.SemaphoreType.DMA,
        pltpu.SemaphoreType.DMA,
        pltpu.SemaphoreType.DMA,
    ],
)(_gather_body)


def kernel(x, table, W, b):
    y = _fold_table(table, W, b)
    out = _gather_call(y, x.reshape(_N))
    return out.reshape(_B, _L, _EMB)


# speedup vs baseline: 1.0019x; 1.0019x over previous
import functools

import jax
import jax.numpy as jnp
from jax import lax
from jax.experimental import pallas as pl
from jax.experimental.pallas import tpu as pltpu
from jax.experimental.pallas import tpu_sc as plsc

_B, _L, _EMB = 4096, 100, 768
_PROMPT = 100
_VPAD = 128
_NC, _NS = 2, 16
_NW = _NC * _NS
_N = _B * _L
_PER_W = _N // _NW
_C = 64
_NCH = _PER_W // _C


def _y_body(t_ref, w_ref, b_ref, y_ref):
    h = lax.dot_general(t_ref[...], w_ref[...],
                        (((1,), (1,)), ((), ())),
                        preferred_element_type=jnp.float32)
    h = h + b_ref[...]
    y_ref[...] = 0.5 * h * (1.0 + lax.erf(h * 0.7071067811865476))


def _fold_table(table, W, b):
    tp = jnp.zeros((_VPAD, _EMB), jnp.float32).at[:_PROMPT, :].set(table)
    return pl.pallas_call(
        _y_body,
        out_shape=jax.ShapeDtypeStruct((_VPAD, _EMB), jnp.float32),
    )(tp, W, b.reshape(1, _EMB))


def _gather_body(y_hbm, x_hbm, out_hbm,
                 idx_v, rows0, rows1, gsem0, gsem1, wsem0, wsem1):
    wid = lax.axis_index("s") * _NC + lax.axis_index("c")
    base = wid * _PER_W

    pltpu.sync_copy(x_hbm.at[pl.ds(base, _PER_W)], idx_v)

    def _idx(c):
        return idx_v.at[pl.ds(c * _C, _C)]

    def _gather(c, rows, sem):
        pltpu.async_copy(y_hbm.at[_idx(c)], rows, sem)

    def _wait_gather(c, rows, sem):
        pltpu.make_async_copy(y_hbm.at[_idx(c)], rows, sem).wait()

    def _dst(c):
        return out_hbm.at[pl.ds(base + c * _C, _C)]

    _gather(0, rows0, gsem0)

    @pl.loop(0, _NCH, step=2)
    def _pair(c):
        @pl.when(c > 0)
        def _():
            pltpu.make_async_copy(rows1, _dst(c - 1), wsem1).wait()

        _gather(c + 1, rows1, gsem1)

        _wait_gather(c, rows0, gsem0)
        pltpu.async_copy(rows0, _dst(c), wsem0)

        _wait_gather(c + 1, rows1, gsem1)
        pltpu.async_copy(rows1, _dst(c + 1), wsem1)

        pltpu.make_async_copy(rows0, _dst(c), wsem0).wait()

        @pl.when(c + 2 < _NCH)
        def _():
            _gather(c + 2, rows0, gsem0)

    pltpu.make_async_copy(rows1, _dst(_NCH - 1), wsem1).wait()


_gather_call = functools.partial(
    pl.kernel,
    out_type=jax.ShapeDtypeStruct((_N, _EMB), jnp.float32),
    mesh=plsc.VectorSubcoreMesh(core_axis_name="c", subcore_axis_name="s"),
    scratch_types=[
        pltpu.VMEM((_PER_W,), jnp.int32),
        pltpu.VMEM((_C, _EMB), jnp.float32),
        pltpu.VMEM((_C, _EMB), jnp.float32),
        pltpu.SemaphoreType.DMA,
        pltpu.SemaphoreType.DMA,
        pltpu.SemaphoreType.DMA,
        pltpu.SemaphoreType.DMA,
    ],
)(_gather_body)


def kernel(x, table, W, b):
    y = _fold_table(table, W, b)
    out = _gather_call(y, x.reshape(_N))
    return out.reshape(_B, _L, _EMB)


# speedup vs baseline: 1.2465x; 1.2441x over previous
import functools

import jax
import jax.numpy as jnp
from jax import lax
from jax.experimental import pallas as pl
from jax.experimental.pallas import tpu as pltpu
from jax.experimental.pallas import tpu_sc as plsc

_B, _L, _EMB = 4096, 100, 768
_PROMPT = 100
_VPAD = 128
_NC, _NS = 2, 16
_NW = _NC * _NS
_BROWS = _B // _NW
_H0 = 48
_H1 = 64
_O1 = 32
_RS = _H0 + _H1
_HT = 4
_OT = 96
_RSP = 120


def _y_body(t_ref, w_ref, b_ref, y_ref):
    h = lax.dot_general(t_ref[...], w_ref[...],
                        (((1,), (1,)), ((), ())),
                        preferred_element_type=jnp.float32)
    h = h + b_ref[...]
    y_ref[...] = 0.5 * h * (1.0 + lax.erf(h * 0.7071067811865476))


def _fold_table(table, W, b):
    tp = jnp.zeros((_VPAD, _EMB), jnp.float32).at[:_PROMPT, :].set(table)
    return pl.pallas_call(
        _y_body,
        out_shape=jax.ShapeDtypeStruct((_VPAD, _EMB), jnp.float32),
    )(tp, W, b.reshape(1, _EMB))


def _gather_body(y_hbm, x_hbm, out_hbm,
                 idx_v, rows0, rows1, rowst,
                 gsem0, gsem1, wsem0, wsem1, tgsem, twsem):
    wid = lax.axis_index("s") * _NC + lax.axis_index("c")
    b0 = wid * _BROWS

    pltpu.sync_copy(x_hbm.at[pl.ds(b0 * _RSP, _BROWS * _RSP)], idx_v)

    def _src0(r):
        return y_hbm.at[idx_v.at[pl.ds(r * _RSP, _H0)]]

    def _src1(r):
        return y_hbm.at[idx_v.at[pl.ds(r * _RSP + _H0, _H1)]]

    def _srct(r):
        return y_hbm.at[idx_v.at[pl.ds(r * _RSP + _H0 + _H1, _HT)]]

    def _dst0(r):
        return out_hbm.at[b0 + r, pl.ds(0, _H0), :]

    def _dst1(r):
        return out_hbm.at[b0 + r, pl.ds(_O1, _H1), :]

    def _dstt(r):
        return out_hbm.at[b0 + r, pl.ds(_OT, _HT), :]

    pltpu.async_copy(_src0(0), rows0, gsem0)

    @pl.loop(0, _BROWS)
    def _row(r):
        @pl.when(r > 0)
        def _():
            pltpu.make_async_copy(rows1, _dst1(r - 1), wsem1).wait()
            pltpu.make_async_copy(rowst, _dstt(r - 1), twsem).wait()

        pltpu.async_copy(_src1(r), rows1, gsem1)
        pltpu.async_copy(_srct(r), rowst, tgsem)

        pltpu.make_async_copy(_src0(r), rows0, gsem0).wait()
        pltpu.async_copy(rows0, _dst0(r), wsem0)

        pltpu.make_async_copy(_src1(r), rows1, gsem1).wait()
        pltpu.async_copy(rows1, _dst1(r), wsem1)

        pltpu.make_async_copy(_srct(r), rowst, tgsem).wait()
        pltpu.async_copy(rowst, _dstt(r), twsem)

        pltpu.make_async_copy(rows0, _dst0(r), wsem0).wait()

        @pl.when(r + 1 < _BROWS)
        def _():
            pltpu.async_copy(_src0(r + 1), rows0, gsem0)

    pltpu.make_async_copy(rows1, _dst1(_BROWS - 1), wsem1).wait()
    pltpu.make_async_copy(rowst, _dstt(_BROWS - 1), twsem).wait()


_gather_call = functools.partial(
    pl.kernel,
    out_type=jax.ShapeDtypeStruct((_B, _L, _EMB), jnp.float32),
    mesh=plsc.VectorSubcoreMesh(core_axis_name="c", subcore_axis_name="s"),
    scratch_types=[
        pltpu.VMEM((_BROWS * _RSP,), jnp.int32),
        pltpu.VMEM((_H0, _EMB), jnp.float32),
        pltpu.VMEM((_H1, _EMB), jnp.float32),
        pltpu.VMEM((_HT, _EMB), jnp.float32),
        pltpu.SemaphoreType.DMA,
        pltpu.SemaphoreType.DMA,
        pltpu.SemaphoreType.DMA,
        pltpu.SemaphoreType.DMA,
        pltpu.SemaphoreType.DMA,
        pltpu.SemaphoreType.DMA,
    ],
)(_gather_body)


def kernel(x, table, W, b):
    y = _fold_table(table, W, b)
    xp = jnp.concatenate(
        [x[:, :_H0], x[:, _O1:_O1 + _H1], x[:, _OT:],
         jnp.zeros((_B, _RSP - _RS - _HT), jnp.int32)],
        axis=1).reshape(_B * _RSP)
    return _gather_call(y, xp)
